# Initial kernel scaffold; baseline (speedup 1.0000x reference)
import jax
import jax.numpy as jnp
from jax.experimental import pallas as pl


def kernel(x, W1, b1, W2, b2):
    raise NotImplementedError("write your pallas kernel here")


# speedup vs baseline: 16.2947x; 16.2947x over previous
import jax
import jax.numpy as jnp
from jax.experimental import pallas as pl
from jax.experimental.pallas import tpu as pltpu

N = 512
H = 128
BI = 64
BJ = 128


def _elu(z):
    return jnp.where(z > 0, z, jnp.exp(z) - 1.0)


def _adj_kernel(x_ref, w1_ref, b1_ref, w2_ref, b2_ref, out_ref, u_s, v_s):
    ti = pl.program_id(0)
    tj = pl.program_id(1)

    @pl.when(jnp.logical_and(ti == 0, tj == 0))
    def _init():
        x = x_ref[...]
        w1 = w1_ref[...]
        dn = (((1,), (1,)), ((), ()))
        u = jax.lax.dot_general(x, w1[:, :H], dn,
                                preferred_element_type=jnp.float32)
        v = jax.lax.dot_general(x, w1[:, H:], dn,
                                preferred_element_type=jnp.float32)
        u_s[...] = u + b1_ref[...]
        v_s[...] = v

    w2 = w2_ref[...].reshape(1, 1, H)
    b2 = b2_ref[0, 0]

    def scores(a_blk, b_blk):
        pre = a_blk[:, None, :] + b_blk[None, :, :]
        return jnp.sum(_elu(pre) * w2, axis=-1)

    ub = u_s[pl.ds(ti * BI, BI), :]
    vb = v_s[pl.ds(ti * BI, BI), :]
    uj = u_s[pl.ds(tj * BJ, BJ), :]
    vj = v_s[pl.ds(tj * BJ, BJ), :]

    purely_upper = (ti + 1) * BI <= tj * BJ
    purely_lower = ti * BI >= (tj + 1) * BJ

    @pl.when(purely_upper)
    def _upper():
        out_ref[...] = scores(ub, vj) + b2

    @pl.when(purely_lower)
    def _lower():
        out_ref[...] = scores(vb, uj) + b2

    @pl.when(jnp.logical_and(jnp.logical_not(purely_upper),
                             jnp.logical_not(purely_lower)))
    def _mixed():
        r = ti * BI + jax.lax.broadcasted_iota(jnp.int32, (BI, BJ), 0)
        c = tj * BJ + jax.lax.broadcasted_iota(jnp.int32, (BI, BJ), 1)
        t1 = scores(ub, vj) + b2
        t2 = scores(vb, uj) + b2
        out_ref[...] = jnp.where(r < c, t1, 0.0) + jnp.where(r > c, t2, 0.0)


def kernel(x, W1, b1, W2, b2):
    b1r = b1.reshape(1, H)
    b2r = b2.reshape(1, 1)
    grid = (N // BI, N // BJ)
    return pl.pallas_call(
        _adj_kernel,
        grid=grid,
        in_specs=[
            pl.BlockSpec((N, H), lambda i, j: (0, 0)),
            pl.BlockSpec((H, 2 * H), lambda i, j: (0, 0)),
            pl.BlockSpec((1, H), lambda i, j: (0, 0)),
            pl.BlockSpec((1, H), lambda i, j: (0, 0)),
            pl.BlockSpec((1, 1), lambda i, j: (0, 0)),
        ],
        out_specs=pl.BlockSpec((BI, BJ), lambda i, j: (i, j)),
        out_shape=jax.ShapeDtypeStruct((N, N), jnp.float32),
        scratch_shapes=[pltpu.VMEM((N, H), jnp.float32),
                        pltpu.VMEM((N, H), jnp.float32)],
    )(x, W1, b1r, W2, b2r)


# speedup vs baseline: 31.4944x; 1.9328x over previous
import jax
import jax.numpy as jnp
from jax.experimental import pallas as pl
from jax.experimental.pallas import tpu as pltpu

N = 512
H = 128
B = 128
NT = N // B


def _elu(z):
    return jnp.where(z > 0, z, jnp.exp(z) - 1.0)


def _adj_kernel(x_ref, w1_ref, b1_ref, w2_ref, b2_ref, out_ref, u_s, v_s):
    ti = pl.program_id(0)
    tj = pl.program_id(1)

    @pl.when(jnp.logical_and(ti == 0, tj == 0))
    def _init():
        x = x_ref[...]
        w1 = w1_ref[...]
        dn = (((1,), (1,)), ((), ()))
        u = jax.lax.dot_general(x, w1[:, :H], dn,
                                preferred_element_type=jnp.float32)
        v = jax.lax.dot_general(x, w1[:, H:], dn,
                                preferred_element_type=jnp.float32)
        u_s[...] = u + b1_ref[...]
        v_s[...] = v

    w2 = w2_ref[...].reshape(1, 1, H)
    b2 = b2_ref[0, 0]

    def scores(a_blk, b_blk):
        pre = a_blk[:, None, :] + b_blk[None, :, :]
        return jnp.sum(_elu(pre) * w2, axis=-1)

    @pl.when(ti < tj)
    def _upper():
        ub = u_s[pl.ds(ti * B, B), :]
        vj = v_s[pl.ds(tj * B, B), :]
        s = scores(ub, vj) + b2
        out_ref[pl.ds(ti * B, B), pl.ds(tj * B, B)] = s
        out_ref[pl.ds(tj * B, B), pl.ds(ti * B, B)] = s.T

    @pl.when(ti == tj)
    def _diag():
        ub = u_s[pl.ds(ti * B, B), :]
        vj = v_s[pl.ds(tj * B, B), :]
        s = scores(ub, vj) + b2
        r = jax.lax.broadcasted_iota(jnp.int32, (B, B), 0)
        c = jax.lax.broadcasted_iota(jnp.int32, (B, B), 1)
        su = jnp.where(r < c, s, 0.0)
        out_ref[pl.ds(ti * B, B), pl.ds(tj * B, B)] = su + su.T


def kernel(x, W1, b1, W2, b2):
    b1r = b1.reshape(1, H)
    b2r = b2.reshape(1, 1)
    return pl.pallas_call(
        _adj_kernel,
        grid=(NT, NT),
        in_specs=[
            pl.BlockSpec((N, H), lambda i, j: (0, 0)),
            pl.BlockSpec((H, 2 * H), lambda i, j: (0, 0)),
            pl.BlockSpec((1, H), lambda i, j: (0, 0)),
            pl.BlockSpec((1, H), lambda i, j: (0, 0)),
            pl.BlockSpec((1, 1), lambda i, j: (0, 0)),
        ],
        out_specs=pl.BlockSpec((N, N), lambda i, j: (0, 0)),
        out_shape=jax.ShapeDtypeStruct((N, N), jnp.float32),
        scratch_shapes=[pltpu.VMEM((N, H), jnp.float32),
                        pltpu.VMEM((N, H), jnp.float32)],
    )(x, W1, b1r, W2, b2r)


# speedup vs baseline: 31.9857x; 1.0156x over previous
import jax
import jax.numpy as jnp
from jax.experimental import pallas as pl
from jax.experimental.pallas import tpu as pltpu

N = 512
H = 128
B = 128
NT = N // B


def _elu(z):
    return jnp.where(z > 0, z, jnp.exp(z) - 1.0)


def _adj_kernel(x_ref, w1_ref, b1_ref, w2_ref, b2_ref, out_ref, u_s, v_s):
    ti = pl.program_id(0)
    tj = pl.program_id(1)

    @pl.when(jnp.logical_and(ti == 0, tj == 0))
    def _init():
        x = x_ref[...]
        w1 = w1_ref[...]
        dn = (((1,), (1,)), ((), ()))
        u = jax.lax.dot_general(x, w1[:, :H], dn,
                                preferred_element_type=jnp.float32)
        v = jax.lax.dot_general(x, w1[:, H:], dn,
                                preferred_element_type=jnp.float32)
        u_s[...] = u + b1_ref[...]
        v_s[...] = v

    w2col = w2_ref[...].reshape(H, 1)
    b2 = b2_ref[0, 0]

    def scores(a_blk, b_blk):
        pre = a_blk[:, None, :] + b_blk[None, :, :]
        act = _elu(pre).reshape(B * B, H)
        s = jax.lax.dot_general(act, w2col, (((1,), (0,)), ((), ())),
                                preferred_element_type=jnp.float32)
        return s.reshape(B, B)

    @pl.when(ti < tj)
    def _upper():
        ub = u_s[pl.ds(ti * B, B), :]
        vj = v_s[pl.ds(tj * B, B), :]
        s = scores(ub, vj) + b2
        out_ref[pl.ds(ti * B, B), pl.ds(tj * B, B)] = s
        out_ref[pl.ds(tj * B, B), pl.ds(ti * B, B)] = s.T

    @pl.when(ti == tj)
    def _diag():
        ub = u_s[pl.ds(ti * B, B), :]
        vj = v_s[pl.ds(tj * B, B), :]
        s = scores(ub, vj) + b2
        r = jax.lax.broadcasted_iota(jnp.int32, (B, B), 0)
        c = jax.lax.broadcasted_iota(jnp.int32, (B, B), 1)
        su = jnp.where(r < c, s, 0.0)
        out_ref[pl.ds(ti * B, B), pl.ds(tj * B, B)] = su + su.T


def kernel(x, W1, b1, W2, b2):
    b1r = b1.reshape(1, H)
    b2r = b2.reshape(1, 1)
    return pl.pallas_call(
        _adj_kernel,
        grid=(NT, NT),
        in_specs=[
            pl.BlockSpec((N, H), lambda i, j: (0, 0)),
            pl.BlockSpec((H, 2 * H), lambda i, j: (0, 0)),
            pl.BlockSpec((1, H), lambda i, j: (0, 0)),
            pl.BlockSpec((1, H), lambda i, j: (0, 0)),
            pl.BlockSpec((1, 1), lambda i, j: (0, 0)),
        ],
        out_specs=pl.BlockSpec((N, N), lambda i, j: (0, 0)),
        out_shape=jax.ShapeDtypeStruct((N, N), jnp.float32),
        scratch_shapes=[pltpu.VMEM((N, H), jnp.float32),
                        pltpu.VMEM((N, H), jnp.float32)],
    )(x, W1, b1r, W2, b2r)
